# Initial kernel scaffold; baseline (speedup 1.0000x reference)
import jax
import jax.numpy as jnp
from jax.experimental import pallas as pl


def kernel(x):
    raise NotImplementedError("write your pallas kernel here")


# speedup vs baseline: 28.0974x; 28.0974x over previous
import jax
import jax.numpy as jnp
from jax import lax
from jax.experimental import pallas as pl
from jax.experimental.pallas import tpu as pltpu
from jax.experimental.pallas import tpu_sc as plsc

_B, _C, _H, _W = 32, 384, 28, 28
_HW = _H * _W
_K = 4
_CG = _C // _K
_S = 24
_NSL = _CG // _S
_CH = _S * _HW
_NV = _CH // 16


def _sc_body(x_hbm, out_hbm, b0, b1, b2, b3, sem):
    cid = lax.axis_index("c")
    sid = lax.axis_index("s")
    wid = sid * 2 + cid
    base = wid * (_C * _HW)
    bufs = (b0, b1, b2, b3)

    for s in range(_NSL):
        offs = [base + (j * _CG + s * _S) * _HW for j in range(_K)]
        cps = [
            pltpu.make_async_copy(x_hbm.at[pl.ds(offs[j], _CH)], bufs[j], sem)
            for j in range(_K)
        ]
        for cp in cps:
            cp.start()
        for cp in cps:
            cp.wait()

        def body(i, carry):
            sl = pl.ds(i * 16, 16)
            a = b0[sl]
            b = b1[sl]
            c = b2[sl]
            d = b3[sl]
            lo1 = jnp.minimum(a, b)
            hi1 = jnp.maximum(a, b)
            lo2 = jnp.minimum(c, d)
            hi2 = jnp.maximum(c, d)
            b0[sl] = jnp.maximum(hi1, hi2)
            b3[sl] = jnp.minimum(lo1, lo2)
            m1 = jnp.minimum(hi1, hi2)
            m2 = jnp.maximum(lo1, lo2)
            b1[sl] = jnp.maximum(m1, m2)
            b2[sl] = jnp.minimum(m1, m2)
            return carry

        lax.fori_loop(0, _NV, body, 0)

        ocps = [
            pltpu.make_async_copy(bufs[j], out_hbm.at[pl.ds(offs[j], _CH)], sem)
            for j in range(_K)
        ]
        for cp in ocps:
            cp.start()
        for cp in ocps:
            cp.wait()


def kernel(x):
    mesh = plsc.VectorSubcoreMesh(core_axis_name="c", subcore_axis_name="s")
    kfn = pl.kernel(
        _sc_body,
        mesh=mesh,
        out_type=jax.ShapeDtypeStruct((_B * _C * _HW,), jnp.float32),
        scratch_types=[pltpu.VMEM((_CH,), jnp.float32) for _ in range(_K)]
        + [pltpu.SemaphoreType.DMA],
    )
    out = kfn(x.reshape(-1))
    return out.reshape(_B, _C, _H, _W)
